# baseline (speedup 1.0000x reference)
import functools

import jax
import jax.numpy as jnp
from jax.experimental import pallas as pl

NHEADS = 4
KS = 64
ROW_BLOCK = 512
N_BISECT = 10
INT_MIN_PY = -2**31
XMASK_PY = 0x7FFFFFFF


def _qkv_kernel(x_ref, wq_ref, bq_ref, wk_ref, bk_ref, wv_ref, bv_ref,
                q_ref, k_ref, v_ref, *, scale):
    x = x_ref[...]
    q_ref[0] = (jnp.dot(x, wq_ref[0], preferred_element_type=jnp.float32)
                + bq_ref[0]) * scale
    k_ref[0] = jnp.dot(x, wk_ref[0],
                       preferred_element_type=jnp.float32) + bk_ref[0]
    v_ref[0] = jnp.dot(x, wv_ref[0],
                       preferred_element_type=jnp.float32) + bv_ref[0]


def _attn_kernel(q_ref, k_ref, v_ref, wp_ref, bp_ref, out_ref):
    h = pl.program_id(1)
    INT_MIN = jnp.int32(INT_MIN_PY)
    XMASK = jnp.int32(XMASK_PY)
    d = jnp.dot(q_ref[0], k_ref[0].T, preferred_element_type=jnp.float32)

    ib = jax.lax.bitcast_convert_type(d, jnp.int32)
    keys = jnp.where(ib >= 0, ib, ib ^ XMASK)

    cm = jnp.max(keys.reshape(ROW_BLOCK, KS, 2048 // KS), axis=2)
    lo = jnp.min(cm, axis=1, keepdims=True)
    m1 = jnp.max(cm, axis=1, keepdims=True)
    tie = jnp.sum((cm == m1).astype(jnp.int32), axis=1, keepdims=True) >= 2
    m2 = jnp.max(jnp.where(cm == m1, INT_MIN, cm), axis=1, keepdims=True)
    hi = jnp.where(tie, m1, m2) + 1

    zero = jnp.zeros((ROW_BLOCK, 1), jnp.int32)
    cl = zero + 2048
    ch = zero
    cl_t = zero
    ch_t = zero

    def count_ge(mid):
        return jnp.sum((keys >= mid).astype(jnp.int32), axis=1, keepdims=True)

    def bis_body(it, carry):
        lo, cl, cl_t, hi, ch, ch_t = carry
        mid = (lo & hi) + ((lo ^ hi) >> 1)
        stale = jnp.logical_and(it == N_BISECT - 1,
                                jnp.logical_and(cl_t == 0, ch_t == 0))
        mid = jnp.maximum(jnp.where(stale, hi - 1, mid), lo)
        cnt = count_ge(mid)
        ge = cnt >= KS
        eq = cnt == KS
        lo = jnp.where(ge, mid, lo)
        cl = jnp.where(ge, cnt, cl)
        cl_t = jnp.where(ge, 1, cl_t)
        hi = jnp.where(eq, mid, jnp.where(ge, hi, mid))
        ch = jnp.where(eq, KS, jnp.where(ge, ch, cnt))
        ch_t = jnp.where(jnp.logical_or(jnp.logical_not(ge), eq), 1, ch_t)
        return lo, cl, cl_t, hi, ch, ch_t

    carry = jax.lax.fori_loop(0, N_BISECT, bis_body,
                              (lo, cl, cl_t, hi, ch, ch_t))

    def row_done(c):
        lo, cl, cl_t, hi, ch, ch_t = c
        return jnp.logical_or(
            jnp.logical_or(jnp.logical_and(cl == KS, cl_t == 1),
                           jnp.logical_and(ch == KS, ch_t == 1)),
            hi <= lo + 1)

    def ext_cond(c):
        return jnp.any(jnp.logical_not(row_done(c)))

    def ext_body(c):
        lo, cl, cl_t, hi, ch, ch_t = c
        active = jnp.logical_not(row_done(c))
        excl = jnp.logical_and(
            cl_t == 1,
            jnp.logical_or(ch_t == 0, cl - KS <= KS - ch))
        flip = jnp.where(excl, jnp.int32(-1), jnp.int32(0))
        x = keys ^ flip
        bound = jnp.where(excl, ~lo, hi - 1)
        m = jnp.max(jnp.where(x <= bound, x, INT_MIN), axis=1, keepdims=True)
        inc_u = jnp.logical_and(active, jnp.logical_not(excl))
        exc_u = jnp.logical_and(active, excl)
        hi = jnp.where(inc_u, m, hi)
        ch = jnp.where(inc_u, ch + 1, ch)
        ch_t = jnp.where(inc_u, 1, ch_t)
        lo = jnp.where(exc_u, ~m + 1, lo)
        cl = jnp.where(exc_u, cl - 1, cl)
        return lo, cl, cl_t, hi, ch, ch_t

    lo, cl, cl_t, hi, ch, ch_t = jax.lax.while_loop(ext_cond, ext_body, carry)

    t = jnp.where(jnp.logical_and(cl == KS, cl_t == 1), lo,
                  jnp.where(jnp.logical_and(ch == KS, ch_t == 1), hi, lo))

    mi = jnp.where(m1 >= 0, m1, m1 ^ XMASK)
    rowmax = jax.lax.bitcast_convert_type(mi, jnp.float32)
    e = jnp.where(keys >= t, jnp.exp(d - rowmax), 0.0)
    acc = jnp.dot(e, v_ref[0], preferred_element_type=jnp.float32)
    head_out = acc / jnp.sum(e, axis=1, keepdims=True)
    contrib = jnp.dot(head_out, wp_ref[0], preferred_element_type=jnp.float32)

    @pl.when(h == 0)
    def _():
        out_ref[...] = contrib + bp_ref[...]

    @pl.when(h != 0)
    def _():
        out_ref[...] += contrib


def kernel(vid, Wq, bq, Wk, bk, Wv, bv, Wp, bp):
    Bv, Tv, Cv, Hv, Wd = vid.shape
    N = Tv * Hv * Wd
    dh = Cv // NHEADS
    scale = dh ** -0.5
    x = vid.transpose(0, 1, 3, 4, 2).reshape(N, Cv)

    def col_heads(w):
        return w.reshape(Cv, NHEADS, dh).transpose(1, 0, 2)

    def bias_heads(b):
        return b.reshape(NHEADS, 1, dh)

    q, k, v = pl.pallas_call(
        functools.partial(_qkv_kernel, scale=scale),
        grid=(NHEADS,),
        in_specs=[
            pl.BlockSpec((N, Cv), lambda h: (0, 0)),
            pl.BlockSpec((1, Cv, dh), lambda h: (h, 0, 0)),
            pl.BlockSpec((1, 1, dh), lambda h: (h, 0, 0)),
            pl.BlockSpec((1, Cv, dh), lambda h: (h, 0, 0)),
            pl.BlockSpec((1, 1, dh), lambda h: (h, 0, 0)),
            pl.BlockSpec((1, Cv, dh), lambda h: (h, 0, 0)),
            pl.BlockSpec((1, 1, dh), lambda h: (h, 0, 0)),
        ],
        out_specs=[pl.BlockSpec((1, N, dh), lambda h: (h, 0, 0))] * 3,
        out_shape=[jax.ShapeDtypeStruct((NHEADS, N, dh), jnp.float32)] * 3,
    )(x, col_heads(Wq), bias_heads(bq), col_heads(Wk), bias_heads(bk),
      col_heads(Wv), bias_heads(bv))

    nrb = N // ROW_BLOCK
    out = pl.pallas_call(
        _attn_kernel,
        grid=(nrb, NHEADS),
        in_specs=[
            pl.BlockSpec((1, ROW_BLOCK, dh), lambda rb, h: (h, rb, 0)),
            pl.BlockSpec((1, N, dh), lambda rb, h: (h, 0, 0)),
            pl.BlockSpec((1, N, dh), lambda rb, h: (h, 0, 0)),
            pl.BlockSpec((1, dh, Cv), lambda rb, h: (h, 0, 0)),
            pl.BlockSpec((1, Cv), lambda rb, h: (0, 0)),
        ],
        out_specs=pl.BlockSpec((ROW_BLOCK, Cv), lambda rb, h: (rb, 0)),
        out_shape=jax.ShapeDtypeStruct((N, Cv), jnp.float32),
    )(q, k, v, Wp.reshape(NHEADS, dh, Cv), bp.reshape(1, Cv))

    return out.reshape(Bv, Tv, Hv, Wd, Cv).transpose(0, 1, 4, 2, 3)
